# initial kernel scaffold (unmeasured)
import jax
import jax.numpy as jnp
from jax import lax
from jax.experimental import pallas as pl
from jax.experimental.pallas import tpu as pltpu

N_DEV = 4


def kernel(x, w_mat, scale_x, scale_w):
    m_per, k = x.shape
    _, n_per = w_mat.shape
    half = m_per // 2

    def body(x_ref, w_ref, sx_ref, sw_ref, out_ref,
             buf_l, buf_r, buf_d, send_sems, recv_sems):
        me = lax.axis_index("i")
        left = lax.rem(me + N_DEV - 1, N_DEV)
        right = lax.rem(me + 1, N_DEV)

        barrier_sem = pltpu.get_barrier_semaphore()
        for nbr in (left, right):
            pl.semaphore_signal(barrier_sem, inc=1, device_id=(nbr,),
                                device_id_type=pl.DeviceIdType.MESH)
        pl.semaphore_wait(barrier_sem, 2)

        p1r = pltpu.make_async_remote_copy(
            src_ref=x_ref, dst_ref=buf_l,
            send_sem=send_sems.at[0], recv_sem=recv_sems.at[0],
            device_id=(right,), device_id_type=pl.DeviceIdType.MESH)
        p1l = pltpu.make_async_remote_copy(
            src_ref=x_ref, dst_ref=buf_r,
            send_sem=send_sems.at[1], recv_sem=recv_sems.at[1],
            device_id=(left,), device_id_type=pl.DeviceIdType.MESH)
        p1r.start()
        p1l.start()

        scale = sx_ref[0] * sw_ref[0]

        def compute_chunk(chunk_ref, origin):
            acc = lax.dot_general(
                chunk_ref[...], w_ref[...],
                dimension_numbers=(((1,), (0,)), ((), ())),
                preferred_element_type=jnp.float32)
            y = acc * scale
            out_ref[pl.ds(origin * m_per, m_per), :] = y * jax.nn.sigmoid(y)

        compute_chunk(x_ref, me)

        p1r.wait_recv()
        p2r = pltpu.make_async_remote_copy(
            src_ref=buf_l.at[pl.ds(0, half)],
            dst_ref=buf_d.at[pl.ds(0, half)],
            send_sem=send_sems.at[2], recv_sem=recv_sems.at[2],
            device_id=(right,), device_id_type=pl.DeviceIdType.MESH)
        p2r.start()

        p1l.wait_recv()
        p2l = pltpu.make_async_remote_copy(
            src_ref=buf_r.at[pl.ds(half, half)],
            dst_ref=buf_d.at[pl.ds(half, half)],
            send_sem=send_sems.at[3], recv_sem=recv_sems.at[3],
            device_id=(left,), device_id_type=pl.DeviceIdType.MESH)
        p2l.start()

        compute_chunk(buf_l, left)
        compute_chunk(buf_r, right)

        p2r.wait_recv()
        p2l.wait_recv()
        compute_chunk(buf_d, lax.rem(me + 2, N_DEV))

        p1r.wait_send()
        p1l.wait_send()
        p2r.wait_send()
        p2l.wait_send()

    return pl.pallas_call(
        body,
        out_shape=jax.ShapeDtypeStruct((N_DEV * m_per, n_per), jnp.float32),
        in_specs=[
            pl.BlockSpec(memory_space=pltpu.VMEM),
            pl.BlockSpec(memory_space=pltpu.VMEM),
            pl.BlockSpec(memory_space=pltpu.SMEM),
            pl.BlockSpec(memory_space=pltpu.SMEM),
        ],
        out_specs=pl.BlockSpec(memory_space=pltpu.VMEM),
        scratch_shapes=[
            pltpu.VMEM((m_per, k), x.dtype),
            pltpu.VMEM((m_per, k), x.dtype),
            pltpu.VMEM((m_per, k), x.dtype),
            pltpu.SemaphoreType.DMA((4,)),
            pltpu.SemaphoreType.DMA((4,)),
        ],
        compiler_params=pltpu.CompilerParams(
            collective_id=0,
            vmem_limit_bytes=128 * 1024 * 1024,
        ),
    )(x, w_mat, scale_x, scale_w)


# baseline (device time: 117946 ns/iter reference)
import jax
import jax.numpy as jnp
from jax import lax
from jax.experimental import pallas as pl
from jax.experimental.pallas import tpu as pltpu

N_DEV = 4
F8 = jnp.float8_e4m3fn


def kernel(x, w_mat, scale_x, scale_w):
    m_per, k = x.shape
    _, n_per = w_mat.shape
    half = m_per // 2
    xt = m_per // 2
    wt = k // 4

    def body(x_hbm, w_hbm, sx_ref, sw_ref, out_hbm,
             x8, buf_l, buf_r, buf_d, w8, stage_x, stage_w, out_stage,
             send_sems, recv_sems, local_sems):
        me = lax.axis_index("i")
        left = lax.rem(me + N_DEV - 1, N_DEV)
        right = lax.rem(me + 1, N_DEV)

        barrier_sem = pltpu.get_barrier_semaphore()
        for nbr in (left, right):
            pl.semaphore_signal(barrier_sem, inc=1, device_id=(nbr,),
                                device_id_type=pl.DeviceIdType.MESH)
        pl.semaphore_wait(barrier_sem, 2)

        for i in range(m_per // xt):
            cp = pltpu.make_async_copy(
                x_hbm.at[pl.ds(i * xt, xt)], stage_x, local_sems.at[0])
            cp.start()
            cp.wait()
            x8[pl.ds(i * xt, xt)] = stage_x[...].astype(F8)

        p1r = pltpu.make_async_remote_copy(
            src_ref=x8, dst_ref=buf_l,
            send_sem=send_sems.at[0], recv_sem=recv_sems.at[0],
            device_id=(right,), device_id_type=pl.DeviceIdType.MESH)
        p1l = pltpu.make_async_remote_copy(
            src_ref=x8, dst_ref=buf_r,
            send_sem=send_sems.at[1], recv_sem=recv_sems.at[1],
            device_id=(left,), device_id_type=pl.DeviceIdType.MESH)
        p1r.start()
        p1l.start()

        for i in range(k // wt):
            cp = pltpu.make_async_copy(
                w_hbm.at[pl.ds(i * wt, wt)], stage_w, local_sems.at[0])
            cp.start()
            cp.wait()
            w8[pl.ds(i * wt, wt)] = stage_w[...].astype(F8)

        scale = sx_ref[0] * sw_ref[0]
        out_cp = [None, None]

        def compute_chunk(chunk_ref, origin, slot):
            if out_cp[slot] is not None:
                out_cp[slot].wait()
            acc = lax.dot_general(
                chunk_ref[...], w8[...],
                dimension_numbers=(((1,), (0,)), ((), ())),
                preferred_element_type=jnp.float32)
            y = acc * scale
            out_stage[slot] = y * jax.nn.sigmoid(y)
            cp = pltpu.make_async_copy(
                out_stage.at[slot],
                out_hbm.at[pl.ds(origin * m_per, m_per)],
                local_sems.at[1 + slot])
            cp.start()
            out_cp[slot] = cp

        compute_chunk(x8, me, 0)

        p1r.wait_recv()
        p2r = pltpu.make_async_remote_copy(
            src_ref=buf_l.at[pl.ds(0, half)],
            dst_ref=buf_d.at[pl.ds(0, half)],
            send_sem=send_sems.at[2], recv_sem=recv_sems.at[2],
            device_id=(right,), device_id_type=pl.DeviceIdType.MESH)
        p2r.start()

        p1l.wait_recv()
        p2l = pltpu.make_async_remote_copy(
            src_ref=buf_r.at[pl.ds(half, half)],
            dst_ref=buf_d.at[pl.ds(half, half)],
            send_sem=send_sems.at[3], recv_sem=recv_sems.at[3],
            device_id=(left,), device_id_type=pl.DeviceIdType.MESH)
        p2l.start()

        compute_chunk(buf_l, left, 1)
        compute_chunk(buf_r, right, 0)

        p2r.wait_recv()
        p2l.wait_recv()
        compute_chunk(buf_d, lax.rem(me + 2, N_DEV), 1)

        out_cp[0].wait()
        out_cp[1].wait()
        p1r.wait_send()
        p1l.wait_send()
        p2r.wait_send()
        p2l.wait_send()

    return pl.pallas_call(
        body,
        out_shape=jax.ShapeDtypeStruct((N_DEV * m_per, n_per), jnp.float32),
        in_specs=[
            pl.BlockSpec(memory_space=pl.ANY),
            pl.BlockSpec(memory_space=pl.ANY),
            pl.BlockSpec(memory_space=pltpu.SMEM),
            pl.BlockSpec(memory_space=pltpu.SMEM),
        ],
        out_specs=pl.BlockSpec(memory_space=pl.ANY),
        scratch_shapes=[
            pltpu.VMEM((m_per, k), F8),
            pltpu.VMEM((m_per, k), F8),
            pltpu.VMEM((m_per, k), F8),
            pltpu.VMEM((m_per, k), F8),
            pltpu.VMEM((k, n_per), F8),
            pltpu.VMEM((m_per // 2, k), jnp.float32),
            pltpu.VMEM((k // 4, n_per), jnp.float32),
            pltpu.VMEM((2, m_per, n_per), jnp.float32),
            pltpu.SemaphoreType.DMA((4,)),
            pltpu.SemaphoreType.DMA((4,)),
            pltpu.SemaphoreType.DMA((4,)),
        ],
        compiler_params=pltpu.CompilerParams(
            collective_id=0,
            vmem_limit_bytes=64 * 1024 * 1024,
        ),
    )(x, w_mat, scale_x, scale_w)


# device time: 112407 ns/iter; 1.0493x vs baseline; 1.0493x over previous
import jax
import jax.numpy as jnp
from jax import lax
from jax.experimental import pallas as pl
from jax.experimental.pallas import tpu as pltpu

N_DEV = 4
F8 = jnp.float8_e4m3fn


def kernel(x, w_mat, scale_x, scale_w):
    m_per, k = x.shape
    _, n_per = w_mat.shape
    half = m_per // 2
    xt = m_per // 4
    n_xt = m_per // xt
    wt = k // 8
    n_wt = k // wt

    def body(x_hbm, w_hbm, sx_ref, sw_ref, out_hbm,
             x8, buf_l, buf_r, buf_d, w8, stage_x, stage_w, out_stage,
             send_sems, recv_sems, prep_sems, out_sems):
        me = lax.axis_index("i")
        left = lax.rem(me + N_DEV - 1, N_DEV)
        right = lax.rem(me + 1, N_DEV)

        cpx = [None, None]
        cpx[0] = pltpu.make_async_copy(
            x_hbm.at[pl.ds(0, xt)], stage_x.at[0], prep_sems.at[0])
        cpx[0].start()

        barrier_sem = pltpu.get_barrier_semaphore()
        for nbr in (left, right):
            pl.semaphore_signal(barrier_sem, inc=1, device_id=(nbr,),
                                device_id_type=pl.DeviceIdType.MESH)
        pl.semaphore_wait(barrier_sem, 2)

        p1r_t, p1l_t = [], []
        for i in range(n_xt):
            s = i % 2
            if i + 1 < n_xt:
                cpx[1 - s] = pltpu.make_async_copy(
                    x_hbm.at[pl.ds((i + 1) * xt, xt)], stage_x.at[1 - s],
                    prep_sems.at[1 - s])
                cpx[1 - s].start()
            cpx[s].wait()
            x8[pl.ds(i * xt, xt)] = stage_x[s].astype(F8)
            tr = pltpu.make_async_remote_copy(
                src_ref=x8.at[pl.ds(i * xt, xt)],
                dst_ref=buf_l.at[pl.ds(i * xt, xt)],
                send_sem=send_sems.at[i], recv_sem=recv_sems.at[i],
                device_id=(right,), device_id_type=pl.DeviceIdType.MESH)
            tr.start()
            p1r_t.append(tr)
            tl = pltpu.make_async_remote_copy(
                src_ref=x8.at[pl.ds(i * xt, xt)],
                dst_ref=buf_r.at[pl.ds(i * xt, xt)],
                send_sem=send_sems.at[n_xt + i], recv_sem=recv_sems.at[n_xt + i],
                device_id=(left,), device_id_type=pl.DeviceIdType.MESH)
            tl.start()
            p1l_t.append(tl)

        cpw = [None, None]
        cpw[0] = pltpu.make_async_copy(
            w_hbm.at[pl.ds(0, wt)], stage_w.at[0], prep_sems.at[0])
        cpw[0].start()
        for i in range(n_wt):
            s = i % 2
            if i + 1 < n_wt:
                cpw[1 - s] = pltpu.make_async_copy(
                    w_hbm.at[pl.ds((i + 1) * wt, wt)], stage_w.at[1 - s],
                    prep_sems.at[1 - s])
                cpw[1 - s].start()
            cpw[s].wait()
            w8[pl.ds(i * wt, wt)] = stage_w[s].astype(F8)

        scale = sx_ref[0] * sw_ref[0]
        out_cp = [None, None]

        def compute_chunk(chunk_ref, origin, slot):
            if out_cp[slot] is not None:
                out_cp[slot].wait()
            acc = lax.dot_general(
                chunk_ref[...], w8[...],
                dimension_numbers=(((1,), (0,)), ((), ())),
                preferred_element_type=jnp.float32)
            y = acc * scale
            out_stage[slot] = y * jax.nn.sigmoid(y)
            cp = pltpu.make_async_copy(
                out_stage.at[slot],
                out_hbm.at[pl.ds(origin * m_per, m_per)],
                out_sems.at[slot])
            cp.start()
            out_cp[slot] = cp

        p1r_t[0].wait_recv()
        p1r_t[1].wait_recv()
        p2r = pltpu.make_async_remote_copy(
            src_ref=buf_l.at[pl.ds(0, half)],
            dst_ref=buf_d.at[pl.ds(0, half)],
            send_sem=send_sems.at[2 * n_xt], recv_sem=recv_sems.at[2 * n_xt],
            device_id=(right,), device_id_type=pl.DeviceIdType.MESH)
        p2r.start()

        compute_chunk(x8, me, 0)

        p1l_t[2].wait_recv()
        p1l_t[3].wait_recv()
        p2l = pltpu.make_async_remote_copy(
            src_ref=buf_r.at[pl.ds(half, half)],
            dst_ref=buf_d.at[pl.ds(half, half)],
            send_sem=send_sems.at[2 * n_xt + 1],
            recv_sem=recv_sems.at[2 * n_xt + 1],
            device_id=(left,), device_id_type=pl.DeviceIdType.MESH)
        p2l.start()

        p1r_t[2].wait_recv()
        p1r_t[3].wait_recv()
        compute_chunk(buf_l, left, 1)
        p1l_t[0].wait_recv()
        p1l_t[1].wait_recv()
        compute_chunk(buf_r, right, 0)

        p2r.wait_recv()
        p2l.wait_recv()
        compute_chunk(buf_d, lax.rem(me + 2, N_DEV), 1)

        out_cp[0].wait()
        out_cp[1].wait()
        for t in p1r_t + p1l_t:
            t.wait_send()
        p2r.wait_send()
        p2l.wait_send()

    return pl.pallas_call(
        body,
        out_shape=jax.ShapeDtypeStruct((N_DEV * m_per, n_per), jnp.float32),
        in_specs=[
            pl.BlockSpec(memory_space=pl.ANY),
            pl.BlockSpec(memory_space=pl.ANY),
            pl.BlockSpec(memory_space=pltpu.SMEM),
            pl.BlockSpec(memory_space=pltpu.SMEM),
        ],
        out_specs=pl.BlockSpec(memory_space=pl.ANY),
        scratch_shapes=[
            pltpu.VMEM((m_per, k), F8),
            pltpu.VMEM((m_per, k), F8),
            pltpu.VMEM((m_per, k), F8),
            pltpu.VMEM((m_per, k), F8),
            pltpu.VMEM((k, n_per), F8),
            pltpu.VMEM((2, m_per // 4, k), jnp.float32),
            pltpu.VMEM((2, k // 8, n_per), jnp.float32),
            pltpu.VMEM((2, m_per, n_per), jnp.float32),
            pltpu.SemaphoreType.DMA((10,)),
            pltpu.SemaphoreType.DMA((10,)),
            pltpu.SemaphoreType.DMA((2,)),
            pltpu.SemaphoreType.DMA((2,)),
        ],
        compiler_params=pltpu.CompilerParams(
            collective_id=0,
            vmem_limit_bytes=64 * 1024 * 1024,
        ),
    )(x, w_mat, scale_x, scale_w)


# device time: 107803 ns/iter; 1.0941x vs baseline; 1.0427x over previous
import jax
import jax.numpy as jnp
from jax import lax
from jax.experimental import pallas as pl
from jax.experimental.pallas import tpu as pltpu

N_DEV = 4
F8 = jnp.float8_e4m3fn


def kernel(x, w_mat, scale_x, scale_w):
    m_per, k = x.shape
    _, n_per = w_mat.shape
    half = m_per // 2
    xt = m_per // 4
    wt = k // 8
    dt = m_per // 4

    T_OFF = [i * xt for i in range(2)]
    B_OFF = [half + i * xt for i in range(2)]
    CAST_ORDER = [T_OFF[0], B_OFF[0], T_OFF[1], B_OFF[1]]

    def body(x_hbm, w_hbm, sx_ref, sw_ref, out_hbm,
             x8, buf_l, buf_r, buf_d, w8, stage_x, stage_w,
             out_stage, out_stage_d, send_sems, recv_sems, prep_sems,
             out_sems):
        me = lax.axis_index("i")
        left = lax.rem(me + N_DEV - 1, N_DEV)
        right = lax.rem(me + 1, N_DEV)

        cpx = [None, None]
        cpx[0] = pltpu.make_async_copy(
            x_hbm.at[pl.ds(CAST_ORDER[0], xt)], stage_x.at[0],
            prep_sems.at[0])
        cpx[0].start()

        barrier_sem = pltpu.get_barrier_semaphore()
        for nbr in (left, right):
            pl.semaphore_signal(barrier_sem, inc=1, device_id=(nbr,),
                                device_id_type=pl.DeviceIdType.MESH)
        pl.semaphore_wait(barrier_sem, 2)

        def p1_send(off, sem_idx, target, dst_buf):
            r = pltpu.make_async_remote_copy(
                src_ref=x8.at[pl.ds(off, xt)],
                dst_ref=dst_buf.at[pl.ds(off, xt)],
                send_sem=send_sems.at[sem_idx], recv_sem=recv_sems.at[sem_idx],
                device_id=(target,), device_id_type=pl.DeviceIdType.MESH)
            r.start()
            return r

        p1 = {}
        for j, off in enumerate(CAST_ORDER):
            s = j % 2
            if j + 1 < len(CAST_ORDER):
                cpx[1 - s] = pltpu.make_async_copy(
                    x_hbm.at[pl.ds(CAST_ORDER[j + 1], xt)],
                    stage_x.at[1 - s], prep_sems.at[1 - s])
                cpx[1 - s].start()
            cpx[s].wait()
            x8[pl.ds(off, xt)] = stage_x[s].astype(F8)
            i = (off % half) // xt
            if off < half:
                p1[("rT", i)] = p1_send(off, i, right, buf_l)
            else:
                p1[("lB", i)] = p1_send(off, 4 + i, left, buf_r)
        for i, off in enumerate(B_OFF):
            p1[("rB", i)] = p1_send(off, 2 + i, right, buf_l)
        for i, off in enumerate(T_OFF):
            p1[("lT", i)] = p1_send(off, 6 + i, left, buf_r)

        def w_tile(i, s):
            cp = pltpu.make_async_copy(
                w_hbm.at[pl.ds(i * wt, wt)], stage_w.at[s], prep_sems.at[s])
            return cp

        def w_cast(i, s):
            w8[pl.ds(i * wt, wt)] = stage_w[s].astype(F8)

        cpw = [None, None]
        cpw[0] = w_tile(0, 0)
        cpw[0].start()

        def run_w(lo, hi):
            for i in range(lo, hi):
                s = i % 2
                if i + 1 < k // wt:
                    cpw[1 - s] = w_tile(i + 1, 1 - s)
                    cpw[1 - s].start()
                cpw[s].wait()
                w_cast(i, s)

        def p2_send(src_buf, off, sem_idx, target):
            r = pltpu.make_async_remote_copy(
                src_ref=src_buf.at[pl.ds(off, dt)],
                dst_ref=buf_d.at[pl.ds(off, dt)],
                send_sem=send_sems.at[sem_idx], recv_sem=recv_sems.at[sem_idx],
                device_id=(target,), device_id_type=pl.DeviceIdType.MESH)
            r.start()
            return r

        run_w(0, 2)
        p1[("rT", 0)].wait_recv()
        p2r0 = p2_send(buf_l, 0, 8, right)
        p1[("lB", 0)].wait_recv()
        p2l0 = p2_send(buf_r, half, 10, left)
        run_w(2, 4)
        p1[("rT", 1)].wait_recv()
        p2r1 = p2_send(buf_l, dt, 9, right)
        p1[("lB", 1)].wait_recv()
        p2l1 = p2_send(buf_r, half + dt, 11, left)
        run_w(4, 8)

        scale = sx_ref[0] * sw_ref[0]
        out_cp = [None, None]
        out_cp_d = [None, None]

        def silu_store(acc, stage_ref, slot_cp, slot, sem, rows, row0):
            y = acc * scale
            stage_ref[slot] = y * jax.nn.sigmoid(y)
            cp = pltpu.make_async_copy(
                stage_ref.at[slot], out_hbm.at[pl.ds(row0, rows)], sem)
            cp.start()
            slot_cp[slot] = cp

        def compute_chunk(chunk_ref, origin, slot):
            if out_cp[slot] is not None:
                out_cp[slot].wait()
            acc = lax.dot_general(
                chunk_ref[...], w8[...],
                dimension_numbers=(((1,), (0,)), ((), ())),
                preferred_element_type=jnp.float32)
            silu_store(acc, out_stage, out_cp, slot, out_sems.at[slot],
                       m_per, origin * m_per)

        def compute_d_sub(off, slot):
            if out_cp_d[slot] is not None:
                out_cp_d[slot].wait()
            acc = lax.dot_general(
                buf_d[pl.ds(off, dt)], w8[...],
                dimension_numbers=(((1,), (0,)), ((), ())),
                preferred_element_type=jnp.float32)
            d_origin = lax.rem(me + 2, N_DEV)
            silu_store(acc, out_stage_d, out_cp_d, slot,
                       out_sems.at[2 + slot], dt, d_origin * m_per + off)

        compute_chunk(x8, me, 0)

        for i in range(2):
            p1[("rB", i)].wait_recv()
        compute_chunk(buf_l, left, 1)
        for i in range(2):
            p1[("lT", i)].wait_recv()
        compute_chunk(buf_r, right, 0)

        p2r0.wait_recv()
        compute_d_sub(0, 0)
        p2l0.wait_recv()
        compute_d_sub(half, 1)
        p2r1.wait_recv()
        compute_d_sub(dt, 0)
        p2l1.wait_recv()
        compute_d_sub(half + dt, 1)

        out_cp[0].wait()
        out_cp[1].wait()
        out_cp_d[0].wait()
        out_cp_d[1].wait()
        for r in p1.values():
            r.wait_send()
        for r in (p2r0, p2r1, p2l0, p2l1):
            r.wait_send()

    return pl.pallas_call(
        body,
        out_shape=jax.ShapeDtypeStruct((N_DEV * m_per, n_per), jnp.float32),
        in_specs=[
            pl.BlockSpec(memory_space=pl.ANY),
            pl.BlockSpec(memory_space=pl.ANY),
            pl.BlockSpec(memory_space=pltpu.SMEM),
            pl.BlockSpec(memory_space=pltpu.SMEM),
        ],
        out_specs=pl.BlockSpec(memory_space=pl.ANY),
        scratch_shapes=[
            pltpu.VMEM((m_per, k), F8),
            pltpu.VMEM((m_per, k), F8),
            pltpu.VMEM((m_per, k), F8),
            pltpu.VMEM((m_per, k), F8),
            pltpu.VMEM((k, n_per), F8),
            pltpu.VMEM((2, m_per // 4, k), jnp.float32),
            pltpu.VMEM((2, k // 8, n_per), jnp.float32),
            pltpu.VMEM((2, m_per, n_per), jnp.float32),
            pltpu.VMEM((2, m_per // 4, n_per), jnp.float32),
            pltpu.SemaphoreType.DMA((12,)),
            pltpu.SemaphoreType.DMA((12,)),
            pltpu.SemaphoreType.DMA((2,)),
            pltpu.SemaphoreType.DMA((4,)),
        ],
        compiler_params=pltpu.CompilerParams(
            collective_id=0,
            vmem_limit_bytes=64 * 1024 * 1024,
        ),
    )(x, w_mat, scale_x, scale_w)
